# baseline (speedup 1.0000x reference)
import functools

import jax
import jax.numpy as jnp
from jax import lax
from jax.experimental import pallas as pl
from jax.experimental.pallas import tpu as pltpu
from jax.experimental.pallas import tpu_sc as plsc

NCLS = 150
STRIDE = 160
HROWS = 96
HCOLS = 256
NBINS = HROWS * HCOLS
NW = 32
NPIX = 8 * 512 * 512
PER_W = NPIX // NW
CHUNK = 8192
NCHUNK = PER_W // CHUNK
LANES = 16
NGROUP = STRIDE // LANES


def _hist_body(preds_hbm, truths_hbm, out_hbm, pbuf, tbuf, hist, slab, stage):
    cid = lax.axis_index("c")
    sid = lax.axis_index("s")
    wid = sid * 2 + cid
    base = wid * PER_W

    zeros = jnp.zeros((LANES,), jnp.int32)
    ones = jnp.ones((LANES,), jnp.int32)

    def zero_row(i, _):
        for u in range(16):
            hist[i, pl.ds(u * LANES, LANES)] = zeros
        return 0

    lax.fori_loop(0, HROWS, zero_row, 0)

    UNROLL = 8

    def chunk_body(c, _):
        off = pl.multiple_of(base + c * CHUNK, CHUNK)
        pltpu.sync_copy(preds_hbm.at[pl.ds(off, CHUNK)], pbuf)
        pltpu.sync_copy(truths_hbm.at[pl.ds(off, CHUNK)], tbuf)

        def vec_body(i, _):
            for u in range(UNROLL):
                b = (i * UNROLL + u) * LANES
                p = pbuf[pl.ds(b, LANES)]
                t = tbuf[pl.ds(b, LANES)]
                idx = t * STRIDE + p
                row = lax.shift_right_logical(idx, 8)
                col = lax.bitwise_and(idx, 255)
                plsc.addupdate_scatter(hist, [row, col], ones)
            return 0

        lax.fori_loop(0, CHUNK // LANES // UNROLL, vec_body, 0)
        return 0

    lax.fori_loop(0, NCHUNK, chunk_body, 0)

    pltpu.sync_copy(hist, stage.at[sid])
    plsc.subcore_barrier()
    srows = 8

    @pl.when(sid < HROWS // srows)
    def _():
        r0 = pl.multiple_of(sid * srows, srows)
        pltpu.sync_copy(stage.at[:, pl.ds(r0, srows)], slab)

        def red_k(k, _):
            for rr in range(srows):
                for u in range(HCOLS // LANES):
                    c = u * LANES
                    slab[0, rr, pl.ds(c, LANES)] = (
                        slab[0, rr, pl.ds(c, LANES)] + slab[k, rr, pl.ds(c, LANES)]
                    )
            return 0

        lax.fori_loop(1, 16, red_k, 0)
        pltpu.sync_copy(slab.at[0], out_hbm.at[cid, pl.ds(r0, srows)])


def _metrics_body(part_hbm, out_hbm, b0, b1, csbuf, rsbuf, dbuf, obuf):
    cid = lax.axis_index("c")
    sid = lax.axis_index("s")

    @pl.when(jnp.logical_and(cid == 0, sid == 0))
    def _():
        pltpu.sync_copy(part_hbm.at[0], b0)
        pltpu.sync_copy(part_hbm.at[1], b1)

        zeros = jnp.zeros((LANES,), jnp.int32)
        iota = lax.iota(jnp.int32, LANES)

        def add_row(r, _):
            for u in range(16):
                c = u * LANES
                b0[r, pl.ds(c, LANES)] = b0[r, pl.ds(c, LANES)] + b1[r, pl.ds(c, LANES)]
            return 0

        lax.fori_loop(0, HROWS, add_row, 0)

        for g in range(NGROUP):
            def cs_body(t, acc):
                flat = t * STRIDE + g * LANES
                r = flat // HCOLS
                c = flat % HCOLS
                return acc + b0[r, pl.ds(c, LANES)]

            accv = lax.fori_loop(0, NCLS, cs_body, zeros)
            csbuf[pl.ds(g * LANES, LANES)] = accv

        for g in range(NGROUP):
            rsbuf[pl.ds(g * LANES, LANES)] = zeros

        def rs_body(t, _):
            flat0 = t * STRIDE
            acc = zeros
            for g in range(NGROUP):
                flat = flat0 + g * LANES
                r = flat // HCOLS
                c = flat % HCOLS
                acc = acc + b0[r, pl.ds(c, LANES)]
            tvec = jnp.full((LANES,), t, jnp.int32)
            plsc.addupdate_scatter(rsbuf, [tvec], acc)
            return 0

        lax.fori_loop(0, NCLS, rs_body, 0)

        for g in range(NGROUP):
            ivec = iota + (g * LANES)
            flat = jnp.minimum(ivec * (STRIDE + 1), NBINS - 1)
            row = lax.shift_right_logical(flat, 8)
            col = lax.bitwise_and(flat, 255)
            dbuf[pl.ds(g * LANES, LANES)] = plsc.load_gather(b0, [row, col])

        fz = jnp.zeros((LANES,), jnp.float32)
        nex_a = fz
        tr_a = fz
        tot_a = fz
        caa_a = fz
        iou_a = fz
        for g in range(NGROUP):
            lanecls = iota + (g * LANES)
            valid = lanecls < NCLS
            cs = csbuf[pl.ds(g * LANES, LANES)].astype(jnp.float32)
            rs = rsbuf[pl.ds(g * LANES, LANES)].astype(jnp.float32)
            d = dbuf[pl.ds(g * LANES, LANES)].astype(jnp.float32)
            exist = jnp.logical_and(cs > 0.0, valid)
            one_f = jnp.ones((LANES,), jnp.float32)
            nex_a = nex_a + jnp.where(exist, one_f, fz)
            tr_a = tr_a + jnp.where(valid, d, fz)
            tot_a = tot_a + jnp.where(valid, rs, fz)
            safe_cs = jnp.where(exist, cs, one_f)
            caa_a = caa_a + jnp.where(exist, d / safe_cs, fz)
            safe_den = jnp.where(exist, cs + rs - d, one_f)
            iou_a = iou_a + jnp.where(exist, d / safe_den, fz)

        nex = jnp.full((LANES,), lax.reduce_sum(nex_a, axes=(0,)), jnp.float32)
        tr = jnp.full((LANES,), lax.reduce_sum(tr_a, axes=(0,)), jnp.float32)
        tot = jnp.full((LANES,), lax.reduce_sum(tot_a, axes=(0,)), jnp.float32)
        caa_s = jnp.full((LANES,), lax.reduce_sum(caa_a, axes=(0,)), jnp.float32)
        iou_s = jnp.full((LANES,), lax.reduce_sum(iou_a, axes=(0,)), jnp.float32)

        miou = iou_s / nex
        tacc = tr / tot
        caa = caa_s / nex
        res = (
            jnp.where(iota == 0, miou, fz)
            + jnp.where(iota == 1, tacc, fz)
            + jnp.where(iota == 2, caa, fz)
        )
        obuf[pl.ds(0, LANES)] = res
        pltpu.sync_copy(obuf, out_hbm)


@jax.jit
def _run(preds_flat, truths_flat):
    mesh = plsc.VectorSubcoreMesh(core_axis_name="c", subcore_axis_name="s")
    part = pl.kernel(
        _hist_body,
        out_type=jax.ShapeDtypeStruct((2, HROWS, HCOLS), jnp.int32),
        mesh=mesh,
        compiler_params=pltpu.CompilerParams(needs_layout_passes=False),
        scratch_types=[
            pltpu.VMEM((CHUNK,), jnp.int32),
            pltpu.VMEM((CHUNK,), jnp.int32),
            pltpu.VMEM((HROWS, HCOLS), jnp.int32),
            pltpu.VMEM((16, 8, HCOLS), jnp.int32),
            pltpu.VMEM_SHARED((16, HROWS, HCOLS), jnp.int32),
        ],
    )(preds_flat, truths_flat)

    out16 = pl.kernel(
        _metrics_body,
        out_type=jax.ShapeDtypeStruct((LANES,), jnp.float32),
        mesh=mesh,
        compiler_params=pltpu.CompilerParams(needs_layout_passes=False),
        scratch_types=[
            pltpu.VMEM((HROWS, HCOLS), jnp.int32),
            pltpu.VMEM((HROWS, HCOLS), jnp.int32),
            pltpu.VMEM((STRIDE,), jnp.int32),
            pltpu.VMEM((STRIDE,), jnp.int32),
            pltpu.VMEM((STRIDE,), jnp.int32),
            pltpu.VMEM((LANES,), jnp.float32),
        ],
    )(part)
    return out16


def kernel(preds, truths):
    out16 = _run(preds.reshape(-1), truths.reshape(-1))
    return out16[:3]
